# baseline (speedup 1.0000x reference)
import jax
import jax.numpy as jnp
from jax import lax
from jax.experimental import pallas as pl
from jax.experimental.pallas import tpu as pltpu
from jax.experimental.pallas import tpu_sc as plsc

N = 10000
E = 320000
D = 128
DH = D // 2

NC = 2
NS = 16
NW = NC * NS
CHUNK = 128
NCHUNK = 80
EPW = NCHUNK * CHUNK
EPAD = NW * EPW
NPAD = N + 16
RPT = 624
RREM = NPAD - NS * RPT


def _half_pass(h_hbm, zeros_hbm, out_hbm, c, s, sidx, didx, buf_a, buf_b,
               agg_sh, sem_a, sem_b):
    roff = pl.multiple_of(s * RPT, 8)
    pltpu.sync_copy(zeros_hbm.at[pl.ds(roff, RPT)],
                    agg_sh.at[pl.ds(roff, RPT)])

    @pl.when(s == NS - 1)
    def _():
        pltpu.sync_copy(zeros_hbm.at[pl.ds(NS * RPT, RREM)],
                        agg_sh.at[pl.ds(NS * RPT, RREM)])

    plsc.subcore_barrier()

    pltpu.async_copy(h_hbm.at[sidx.at[0]], buf_a, sem_a)

    def pair(i, carry):
        ja = 2 * i
        jb = 2 * i + 1
        pltpu.async_copy(h_hbm.at[sidx.at[jb]], buf_b, sem_b)
        pltpu.make_async_copy(h_hbm.at[sidx.at[ja]], buf_a, sem_a).wait()
        pltpu.sync_copy(buf_a, agg_sh.at[didx.at[ja]], add=True)

        @pl.when(jb + 1 < NCHUNK)
        def _():
            pltpu.async_copy(h_hbm.at[sidx.at[jb + 1]], buf_a, sem_a)

        pltpu.make_async_copy(h_hbm.at[sidx.at[jb]], buf_b, sem_b).wait()
        pltpu.sync_copy(buf_b, agg_sh.at[didx.at[jb]], add=True)
        return carry

    lax.fori_loop(0, NCHUNK // 2, pair, 0)
    plsc.subcore_barrier()

    pltpu.sync_copy(agg_sh.at[pl.ds(roff, RPT)],
                    out_hbm.at[c, pl.ds(roff, RPT)])

    @pl.when(s == NS - 1)
    def _():
        pltpu.sync_copy(agg_sh.at[pl.ds(NS * RPT, N - NS * RPT)],
                        out_hbm.at[c, pl.ds(NS * RPT, N - NS * RPT)])

    plsc.subcore_barrier()


def _sc_aggregate_body(h0_hbm, h1_hbm, src_hbm, dst_hbm, zeros_hbm, out_hbm,
                       sidx, didx, buf_a, buf_b, agg_sh, sem_a, sem_b):
    c = lax.axis_index("c")
    s = lax.axis_index("s")
    wid = s * NC + c

    pltpu.sync_copy(src_hbm.at[wid], sidx)
    pltpu.sync_copy(dst_hbm.at[wid], didx)

    _half_pass(h0_hbm, zeros_hbm, out_hbm.at[0], c, s, sidx, didx,
               buf_a, buf_b, agg_sh, sem_a, sem_b)
    _half_pass(h1_hbm, zeros_hbm, out_hbm.at[1], c, s, sidx, didx,
               buf_a, buf_b, agg_sh, sem_a, sem_b)


@jax.jit
def _sc_aggregate(h0, h1, src3, dst3, zeros):
    mesh = plsc.VectorSubcoreMesh(core_axis_name="c", subcore_axis_name="s")
    return pl.kernel(
        _sc_aggregate_body,
        out_type=jax.ShapeDtypeStruct((2, NC, N, DH), jnp.float32),
        mesh=mesh,
        compiler_params=pltpu.CompilerParams(use_tc_tiling_on_sc=False),
        scratch_types=[
            pltpu.VMEM((NCHUNK, CHUNK), jnp.int32),
            pltpu.VMEM((NCHUNK, CHUNK), jnp.int32),
            pltpu.VMEM((CHUNK, DH), jnp.float32),
            pltpu.VMEM((CHUNK, DH), jnp.float32),
            pltpu.VMEM_SHARED((NPAD, DH), jnp.float32),
            pltpu.SemaphoreType.DMA,
            pltpu.SemaphoreType.DMA,
        ],
    )(h0, h1, src3, dst3, zeros)


ROWS_BLK = 400


def _tc_layer_body(h_ref, agg_ref, w_ref, b_ref, o_ref):
    agg = jnp.concatenate(
        [agg_ref[0, 0] + agg_ref[0, 1], agg_ref[1, 0] + agg_ref[1, 1]],
        axis=1)
    hs = h_ref[...] + agg
    acc = jnp.dot(hs, w_ref[...], preferred_element_type=jnp.float32)
    o_ref[...] = acc + b_ref[...]


@jax.jit
def _tc_layer(h, agg, w_t, b_row):
    grid = (N // ROWS_BLK,)
    return pl.pallas_call(
        _tc_layer_body,
        grid=grid,
        in_specs=[
            pl.BlockSpec((ROWS_BLK, D), lambda i: (i, 0)),
            pl.BlockSpec((2, NC, ROWS_BLK, DH), lambda i: (0, 0, i, 0)),
            pl.BlockSpec((D, D), lambda i: (0, 0)),
            pl.BlockSpec((1, D), lambda i: (0, 0)),
        ],
        out_specs=pl.BlockSpec((ROWS_BLK, D), lambda i: (i, 0)),
        out_shape=jax.ShapeDtypeStruct((N, D), jnp.float32),
    )(h, agg, w_t, b_row)


def kernel(x, edge_index, W1, b1):
    src = edge_index[0].astype(jnp.int32)
    dst = edge_index[1].astype(jnp.int32)
    src3 = jnp.concatenate(
        [src, jnp.zeros(EPAD - E, jnp.int32)]).reshape(NW, NCHUNK, CHUNK)
    dst3 = jnp.concatenate(
        [dst, jnp.full(EPAD - E, N, jnp.int32)]).reshape(NW, NCHUNK, CHUNK)
    w_t = W1.T
    b_row = b1.reshape(1, D)
    zeros = jnp.zeros((NPAD, DH), jnp.float32)

    x0, x1 = x[:, :DH], x[:, DH:]
    agg1 = _sc_aggregate(x0, x1, src3, dst3, zeros)
    hid = _tc_layer(x, agg1, w_t, b_row)
    h0, h1 = hid[:, :DH], hid[:, DH:]
    agg2 = _sc_aggregate(h0, h1, src3, dst3, zeros)
    out = _tc_layer(hid, agg2, w_t, b_row)
    return (out, hid)


# speedup vs baseline: 1.0217x; 1.0217x over previous
import jax
import jax.numpy as jnp
from jax import lax
from jax.experimental import pallas as pl
from jax.experimental.pallas import tpu as pltpu
from jax.experimental.pallas import tpu_sc as plsc

N = 10000
E = 320000
D = 128
DH = D // 2

NC = 2
NS = 16
NW = NC * NS
CHUNK = 128
NCHUNK = 80
EPW = NCHUNK * CHUNK
EPAD = NW * EPW
NPAD = N + 16
RPT = 624
RREM = NPAD - NS * RPT


def _half_pass(h_hbm, zeros_hbm, out_hbm, c, s, sidx, didx, buf_a, buf_b,
               agg_sh, sem_a, sem_b):
    roff = pl.multiple_of(s * RPT, 8)
    pltpu.sync_copy(zeros_hbm.at[pl.ds(roff, RPT)],
                    agg_sh.at[pl.ds(roff, RPT)])

    @pl.when(s == NS - 1)
    def _():
        pltpu.sync_copy(zeros_hbm.at[pl.ds(NS * RPT, RREM)],
                        agg_sh.at[pl.ds(NS * RPT, RREM)])

    plsc.subcore_barrier()

    pltpu.async_copy(h_hbm.at[sidx.at[0]], buf_a, sem_a)

    def pair(i, carry):
        ja = 2 * i
        jb = 2 * i + 1
        pltpu.async_copy(h_hbm.at[sidx.at[jb]], buf_b, sem_b)
        pltpu.make_async_copy(h_hbm.at[sidx.at[ja]], buf_a, sem_a).wait()
        pltpu.sync_copy(buf_a, agg_sh.at[didx.at[ja]], add=True)

        @pl.when(jb + 1 < NCHUNK)
        def _():
            pltpu.async_copy(h_hbm.at[sidx.at[jb + 1]], buf_a, sem_a)

        pltpu.make_async_copy(h_hbm.at[sidx.at[jb]], buf_b, sem_b).wait()
        pltpu.sync_copy(buf_b, agg_sh.at[didx.at[jb]], add=True)
        return carry

    lax.fori_loop(0, NCHUNK // 2, pair, 0)
    plsc.subcore_barrier()

    pltpu.sync_copy(agg_sh.at[pl.ds(roff, RPT)],
                    out_hbm.at[c, pl.ds(roff, RPT)])

    @pl.when(s == NS - 1)
    def _():
        pltpu.sync_copy(agg_sh.at[pl.ds(NS * RPT, N - NS * RPT)],
                        out_hbm.at[c, pl.ds(NS * RPT, N - NS * RPT)])

    plsc.subcore_barrier()


def _sc_aggregate_body(h0_hbm, h1_hbm, src_hbm, dst_hbm, zeros_hbm, out_hbm,
                       sidx, didx, buf_a, buf_b, agg_sh, sem_a, sem_b):
    c = lax.axis_index("c")
    s = lax.axis_index("s")
    wid = s * NC + c

    pltpu.sync_copy(src_hbm.at[wid], sidx)
    pltpu.sync_copy(dst_hbm.at[wid], didx)

    _half_pass(h0_hbm, zeros_hbm, out_hbm.at[0], c, s, sidx, didx,
               buf_a, buf_b, agg_sh, sem_a, sem_b)
    _half_pass(h1_hbm, zeros_hbm, out_hbm.at[1], c, s, sidx, didx,
               buf_a, buf_b, agg_sh, sem_a, sem_b)


@jax.jit
def _sc_aggregate(h0, h1, src3, dst3, zeros):
    mesh = plsc.VectorSubcoreMesh(core_axis_name="c", subcore_axis_name="s")
    return pl.kernel(
        _sc_aggregate_body,
        out_type=jax.ShapeDtypeStruct((2, NC, N, DH), jnp.float32),
        mesh=mesh,
        compiler_params=pltpu.CompilerParams(use_tc_tiling_on_sc=False),
        scratch_types=[
            pltpu.VMEM((NCHUNK, CHUNK), jnp.int32),
            pltpu.VMEM((NCHUNK, CHUNK), jnp.int32),
            pltpu.VMEM((CHUNK, DH), jnp.float32),
            pltpu.VMEM((CHUNK, DH), jnp.float32),
            pltpu.VMEM_SHARED((NPAD, DH), jnp.float32),
            pltpu.SemaphoreType.DMA,
            pltpu.SemaphoreType.DMA,
        ],
    )(h0, h1, src3, dst3, zeros)


ROWS_BLK = 400


def _tc_layer_body(h_ref, agg_ref, w_ref, b_ref, o_ref):
    agg = jnp.concatenate(
        [agg_ref[0, 0] + agg_ref[0, 1], agg_ref[1, 0] + agg_ref[1, 1]],
        axis=1)
    hs = h_ref[...] + agg
    acc = jnp.dot(hs, w_ref[...], preferred_element_type=jnp.float32)
    o_ref[...] = acc + b_ref[...]


@jax.jit
def _tc_layer(h, agg, w_t, b_row):
    grid = (N // ROWS_BLK,)
    return pl.pallas_call(
        _tc_layer_body,
        grid=grid,
        in_specs=[
            pl.BlockSpec((ROWS_BLK, D), lambda i: (i, 0)),
            pl.BlockSpec((2, NC, ROWS_BLK, DH), lambda i: (0, 0, i, 0)),
            pl.BlockSpec((D, D), lambda i: (0, 0)),
            pl.BlockSpec((1, D), lambda i: (0, 0)),
        ],
        out_specs=pl.BlockSpec((ROWS_BLK, D), lambda i: (i, 0)),
        out_shape=jax.ShapeDtypeStruct((N, D), jnp.float32),
    )(h, agg, w_t, b_row)


def kernel(x, edge_index, W1, b1):
    src = edge_index[0].astype(jnp.int32)
    dst = edge_index[1].astype(jnp.int32)
    npad_e = EPAD - E
    src2 = jnp.concatenate(
        [src, jnp.zeros(npad_e, jnp.int32)]).reshape(EPW, NW).T
    dst2 = jnp.concatenate(
        [dst, jnp.zeros(npad_e, jnp.int32)]).reshape(EPW, NW).T
    ppw = npad_e // NW
    pad_dst = jnp.broadcast_to(N + (jnp.arange(ppw) % 16), (NW, ppw))
    dst2 = dst2.at[:, EPW - ppw:].set(pad_dst.astype(jnp.int32))
    src3 = src2.reshape(NW, NCHUNK, CHUNK)
    dst3 = dst2.reshape(NW, NCHUNK, CHUNK)
    w_t = W1.T
    b_row = b1.reshape(1, D)
    zeros = jnp.zeros((NPAD, DH), jnp.float32)

    x0, x1 = x[:, :DH], x[:, DH:]
    agg1 = _sc_aggregate(x0, x1, src3, dst3, zeros)
    hid = _tc_layer(x, agg1, w_t, b_row)
    h0, h1 = hid[:, :DH], hid[:, DH:]
    agg2 = _sc_aggregate(h0, h1, src3, dst3, zeros)
    out = _tc_layer(hid, agg2, w_t, b_row)
    return (out, hid)


# speedup vs baseline: 2.5229x; 2.4693x over previous
import jax
import jax.numpy as jnp
from jax import lax
from jax.experimental import pallas as pl
from jax.experimental.pallas import tpu as pltpu
from jax.experimental.pallas import tpu_sc as plsc

N = 10000
E = 320000
D = 128
DH = D // 2

NC = 2
NS = 16
NW = NC * NS
CHUNK = 128
NCHUNK = 80
EPW = NCHUNK * CHUNK
EPAD = NW * EPW
NPADE = EPAD - E
NPAD = N
RPT = 624
RREM = NPAD - NS * RPT


def _half_pass(h_hbm, zeros_hbm, out_hbm, c, s, sidx, didx, buf_a, buf_b,
               agg_sh, sem_a, sem_b):
    roff = pl.multiple_of(s * RPT, 8)
    pltpu.sync_copy(zeros_hbm.at[pl.ds(roff, RPT)],
                    agg_sh.at[pl.ds(roff, RPT)])

    @pl.when(s == NS - 1)
    def _():
        pltpu.sync_copy(zeros_hbm.at[pl.ds(NS * RPT, RREM)],
                        agg_sh.at[pl.ds(NS * RPT, RREM)])

    plsc.subcore_barrier()

    pltpu.async_copy(h_hbm.at[sidx.at[0]], buf_a, sem_a)

    def pair(i, carry):
        ja = 2 * i
        jb = 2 * i + 1
        pltpu.async_copy(h_hbm.at[sidx.at[jb]], buf_b, sem_b)
        pltpu.make_async_copy(h_hbm.at[sidx.at[ja]], buf_a, sem_a).wait()
        pltpu.sync_copy(buf_a, agg_sh.at[didx.at[ja]], add=True)

        @pl.when(jb + 1 < NCHUNK)
        def _():
            pltpu.async_copy(h_hbm.at[sidx.at[jb + 1]], buf_a, sem_a)

        pltpu.make_async_copy(h_hbm.at[sidx.at[jb]], buf_b, sem_b).wait()
        pltpu.sync_copy(buf_b, agg_sh.at[didx.at[jb]], add=True)
        return carry

    lax.fori_loop(0, NCHUNK // 2, pair, 0)
    plsc.subcore_barrier()

    pltpu.sync_copy(agg_sh.at[pl.ds(roff, RPT)],
                    out_hbm.at[c, pl.ds(roff, RPT)])

    @pl.when(s == NS - 1)
    def _():
        pltpu.sync_copy(agg_sh.at[pl.ds(NS * RPT, N - NS * RPT)],
                        out_hbm.at[c, pl.ds(NS * RPT, N - NS * RPT)])

    plsc.subcore_barrier()


def _sc_aggregate_body(h0_hbm, h1_hbm, src_hbm, dst_hbm, zeros_hbm, out_hbm,
                       sidx, didx, buf_a, buf_b, agg_sh, sem_a, sem_b):
    c = lax.axis_index("c")
    s = lax.axis_index("s")
    wid = s * NC + c

    pltpu.sync_copy(src_hbm.at[wid], sidx)
    pltpu.sync_copy(dst_hbm.at[wid], didx)

    _half_pass(h0_hbm, zeros_hbm, out_hbm.at[0], c, s, sidx, didx,
               buf_a, buf_b, agg_sh, sem_a, sem_b)
    _half_pass(h1_hbm, zeros_hbm, out_hbm.at[1], c, s, sidx, didx,
               buf_a, buf_b, agg_sh, sem_a, sem_b)


@jax.jit
def _sc_aggregate(h0, h1, src3, dst3, zeros):
    mesh = plsc.VectorSubcoreMesh(core_axis_name="c", subcore_axis_name="s")
    return pl.kernel(
        _sc_aggregate_body,
        out_type=jax.ShapeDtypeStruct((2, NC, N, DH), jnp.float32),
        mesh=mesh,
        compiler_params=pltpu.CompilerParams(use_tc_tiling_on_sc=False),
        scratch_types=[
            pltpu.VMEM((NCHUNK, CHUNK), jnp.int32),
            pltpu.VMEM((NCHUNK, CHUNK), jnp.int32),
            pltpu.VMEM((CHUNK, DH), jnp.float32),
            pltpu.VMEM((CHUNK, DH), jnp.float32),
            pltpu.VMEM_SHARED((NPAD, DH), jnp.float32),
            pltpu.SemaphoreType.DMA,
            pltpu.SemaphoreType.DMA,
        ],
    )(h0, h1, src3, dst3, zeros)


ROWS_BLK = 400


def _tc_layer_body(h_ref, agg_ref, w_ref, b_ref, o_ref):
    agg = jnp.concatenate(
        [agg_ref[0, 0] + agg_ref[0, 1], agg_ref[1, 0] + agg_ref[1, 1]],
        axis=1)
    row = (pl.program_id(0) * ROWS_BLK
           + jax.lax.broadcasted_iota(jnp.int32, (ROWS_BLK, 1), 0))
    hs = jnp.where(row < NPADE, agg, h_ref[...] + agg)
    acc = jnp.dot(hs, w_ref[...], preferred_element_type=jnp.float32)
    o_ref[...] = acc + b_ref[...]


@jax.jit
def _tc_layer(h, agg, w_t, b_row):
    grid = (N // ROWS_BLK,)
    return pl.pallas_call(
        _tc_layer_body,
        grid=grid,
        in_specs=[
            pl.BlockSpec((ROWS_BLK, D), lambda i: (i, 0)),
            pl.BlockSpec((2, NC, ROWS_BLK, DH), lambda i: (0, 0, i, 0)),
            pl.BlockSpec((D, D), lambda i: (0, 0)),
            pl.BlockSpec((1, D), lambda i: (0, 0)),
        ],
        out_specs=pl.BlockSpec((ROWS_BLK, D), lambda i: (i, 0)),
        out_shape=jax.ShapeDtypeStruct((N, D), jnp.float32),
    )(h, agg, w_t, b_row)


def kernel(x, edge_index, W1, b1):
    src = edge_index[0].astype(jnp.int32)
    dst = edge_index[1].astype(jnp.int32)
    pad_idx = jnp.arange(NPADE, dtype=jnp.int32)
    src3 = jnp.concatenate([src, pad_idx]).reshape(NW, NCHUNK, CHUNK)
    dst3 = jnp.concatenate([dst, pad_idx]).reshape(NW, NCHUNK, CHUNK)
    w_t = W1.T
    b_row = b1.reshape(1, D)
    zeros = jnp.zeros((NPAD, DH), jnp.float32)

    x0, x1 = x[:, :DH], x[:, DH:]
    agg1 = _sc_aggregate(x0, x1, src3, dst3, zeros)
    hid = _tc_layer(x, agg1, w_t, b_row)
    h0, h1 = hid[:, :DH], hid[:, DH:]
    agg2 = _sc_aggregate(h0, h1, src3, dst3, zeros)
    out = _tc_layer(hid, agg2, w_t, b_row)
    return (out, hid)
